# baseline (speedup 1.0000x reference)
import functools
import jax
import jax.numpy as jnp
from jax import lax
from jax.experimental import pallas as pl
from jax.experimental.pallas import tpu as pltpu
from jax.experimental.pallas import tpu_sc as plsc

D = 64
DP = 128
SCALE = 8.0
NC, NS = 2, 16
NW = NC * NS
TB = 128


@functools.lru_cache(maxsize=None)
def _build(R, S, V):
    band = TB * S
    mesh = plsc.VectorSubcoreMesh(core_axis_name="c", subcore_axis_name="s")

    @functools.partial(
        pl.kernel,
        out_type=jax.ShapeDtypeStruct((S, D, R), jnp.float32),
        mesh=mesh,
        compiler_params=pltpu.CompilerParams(
            use_tc_tiling_on_sc=True, needs_layout_passes=False),
        scratch_types=[
            pltpu.VMEM((band,), jnp.int32),
            pltpu.VMEM((TB,), jnp.int32),
            pltpu.VMEM((TB,), jnp.int32),
            pltpu.VMEM((TB, DP), jnp.float32),
            pltpu.VMEM((TB, DP), jnp.float32),
            pltpu.VMEM((D, TB), jnp.float32),
            pltpu.SemaphoreType.DMA,
        ],
    )
    def emb(idx_hbm, table_hbm, out_hbm, band_v, il0, il1, rw0, rw1,
            obuf, gsem):
        wid = lax.axis_index("s") * NC + lax.axis_index("c")
        pltpu.sync_copy(idx_hbm.at[pl.ds(wid * band, band)], band_v)

        lanes = lax.iota(jnp.int32, 16)
        lane_s = lanes * S
        ils = (il0, il1)
        rws = (rw0, rw1)

        def build_and_fire(c, p):
            for g in range(8):
                v = plsc.load_gather(band_v, [lane_s + (g * 16 * S + c)])
                ils[p][pl.ds(g * 16, 16)] = v
            pltpu.async_copy(table_hbm.at[ils[p]], rws[p], gsem)

        def drain(p):
            pltpu.make_async_copy(table_hbm.at[ils[p]], rws[p], gsem).wait()

        def transpose_out(c, p):
            rw = rws[p]

            def dbody(d, carry):
                dv = jnp.full((16,), d, jnp.int32)
                for g in range(8):
                    v = plsc.load_gather(rw, [lanes + g * 16, dv])
                    obuf[d, pl.ds(g * 16, 16)] = v * SCALE
                return carry

            lax.fori_loop(0, D, dbody, 0)
            pltpu.sync_copy(obuf, out_hbm.at[c, pl.ds(0, D), pl.ds(wid * TB, TB)])

        build_and_fire(0, 0)
        build_and_fire(1, 1)

        def pair(c2, carry):
            c = c2 * 2
            for sub in range(2):
                drain(sub)
                transpose_out(c + sub, sub)
                build_and_fire(c + sub + 2, sub)
            return carry

        lax.fori_loop(0, S // 2 - 1, pair, 0)
        for sub in range(2):
            drain(sub)
            transpose_out(S - 2 + sub, sub)

    return emb


def kernel(x, lut):
    R, S = x.shape
    V = lut.shape[0]
    xf = x.reshape(-1).astype(jnp.int32)
    tp = jnp.pad(lut, ((0, 0), (0, DP - D)))
    out = _build(R, S, V)(xf, tp)
    return out.transpose(2, 0, 1)


# speedup vs baseline: 1.0009x; 1.0009x over previous
import functools
import jax
import jax.numpy as jnp
from jax import lax
from jax.experimental import pallas as pl
from jax.experimental.pallas import tpu as pltpu
from jax.experimental.pallas import tpu_sc as plsc

D = 64
DP = 128
SCALE = 8.0
NC, NS = 2, 16
NW = NC * NS
TB = 128


@functools.lru_cache(maxsize=None)
def _build(R, S, V):
    band = TB * S
    mesh = plsc.VectorSubcoreMesh(core_axis_name="c", subcore_axis_name="s")

    @functools.partial(
        pl.kernel,
        out_type=jax.ShapeDtypeStruct((S, D, R), jnp.float32),
        mesh=mesh,
        compiler_params=pltpu.CompilerParams(
            use_tc_tiling_on_sc=True, needs_layout_passes=False),
        scratch_types=[
            pltpu.VMEM((band,), jnp.int32),
            pltpu.VMEM((TB,), jnp.int32),
            pltpu.VMEM((TB,), jnp.int32),
            pltpu.VMEM((TB,), jnp.int32),
            pltpu.VMEM((TB,), jnp.int32),
            pltpu.VMEM((TB, DP), jnp.float32),
            pltpu.VMEM((TB, DP), jnp.float32),
            pltpu.VMEM((D, TB), jnp.float32),
            pltpu.VMEM((D, TB), jnp.float32),
            pltpu.SemaphoreType.DMA,
            pltpu.SemaphoreType.DMA,
        ],
    )
    def emb(idx_hbm, table_hbm, out_hbm, band_v, il0, il1, pc0, pc1,
            rw0, rw1, ob0, ob1, gsem, osem):
        wid = lax.axis_index("s") * NC + lax.axis_index("c")
        pltpu.sync_copy(idx_hbm.at[pl.ds(wid * band, band)], band_v)

        lanes = lax.iota(jnp.int32, 16)
        lane_s = lanes * S
        ils = (il0, il1)
        pcs = (pc0, pc1)
        rws = (rw0, rw1)
        obs = (ob0, ob1)

        def build_and_fire(c, p):
            for g in range(8):
                raw = plsc.load_gather(band_v, [lane_s + (g * 16 * S + c)])
                ils[p][pl.ds(g * 16, 16)] = raw >> 1
                pcs[p][pl.ds(g * 16, 16)] = (raw & 1) * D
            pltpu.async_copy(table_hbm.at[ils[p]], rws[p], gsem)

        def drain_gather(p):
            pltpu.make_async_copy(table_hbm.at[ils[p]], rws[p], gsem).wait()

        def out_copy(c, p):
            return pltpu.make_async_copy(
                obs[p], out_hbm.at[c, pl.ds(0, D), pl.ds(wid * TB, TB)], osem)

        def transpose_out(c, p):
            rw = rws[p]
            ob = obs[p]
            cols = [pcs[p][pl.ds(g * 16, 16)] for g in range(8)]

            def dbody(d4, carry):
                cs = carry
                for u in range(4):
                    d = d4 * 4 + u
                    dv = jnp.full((16,), d, jnp.int32)
                    for g in range(8):
                        v = plsc.load_gather(rw, [lanes + g * 16, cs[g] + dv])
                        ob[d, pl.ds(g * 16, 16)] = v * SCALE
                return cs

            lax.fori_loop(0, D // 4, dbody, tuple(cols))
            out_copy(c, p).start()

        build_and_fire(0, 0)
        build_and_fire(1, 1)

        def pair(c2, carry):
            c = c2 * 2
            for sub in range(2):
                drain_gather(sub)
                @pl.when(c2 > 0)
                def _():
                    out_copy(c + sub - 2, sub).wait()
                transpose_out(c + sub, sub)
                build_and_fire(c + sub + 2, sub)
            return carry

        lax.fori_loop(0, S // 2 - 1, pair, 0)
        for sub in range(2):
            drain_gather(sub)
            out_copy(S - 4 + sub, sub).wait()
            transpose_out(S - 2 + sub, sub)
        for sub in range(2):
            out_copy(S - 2 + sub, sub).wait()

    return emb


def kernel(x, lut):
    R, S = x.shape
    V = lut.shape[0]
    xf = x.reshape(-1).astype(jnp.int32)
    tp = lut.reshape(V // 2, DP)
    out = _build(R, S, V)(xf, tp)
    return out.transpose(2, 0, 1)


# speedup vs baseline: 1.7567x; 1.7552x over previous
import functools
import jax
import jax.numpy as jnp
from jax import lax
from jax.experimental import pallas as pl
from jax.experimental.pallas import tpu as pltpu
from jax.experimental.pallas import tpu_sc as plsc

D = 64
DP = 128
SCALE = 8.0
NC, NS = 2, 16
NW = NC * NS
SPLITS = ((0, 104), (104, 96))


@functools.lru_cache(maxsize=None)
def _build(R, S, V):
    rows_per_w = R // NW
    mesh = plsc.VectorSubcoreMesh(core_axis_name="c", subcore_axis_name="s")

    @functools.partial(
        pl.kernel,
        out_type=jax.ShapeDtypeStruct((R, S, D), jnp.float32),
        mesh=mesh,
        compiler_params=pltpu.CompilerParams(
            use_tc_tiling_on_sc=True, needs_layout_passes=False),
        scratch_types=[
            pltpu.VMEM((S,), jnp.int32),
            pltpu.VMEM((S,), jnp.int32),
            pltpu.VMEM((S, DP), jnp.float32),
            pltpu.VMEM((S, DP), jnp.float32),
            pltpu.VMEM((S, D), jnp.float32),
            pltpu.VMEM((S, D), jnp.float32),
            pltpu.SemaphoreType.DMA,
            pltpu.SemaphoreType.DMA,
        ],
    )
    def emb(idx_hbm, table_hbm, out_hbm, ix0, ix1, rw0, rw1, ob0, ob1,
            gsem, osem):
        wid = lax.axis_index("s") * NC + lax.axis_index("c")
        row0 = wid * rows_per_w
        ixs = (ix0, ix1)
        rws = (rw0, rw1)
        obs = (ob0, ob1)

        def fire(g, p):
            pltpu.sync_copy(idx_hbm.at[row0 + g], ixs[p])
            for off, ln in SPLITS:
                pltpu.async_copy(
                    table_hbm.at[ixs[p].at[pl.ds(off, ln)]],
                    rws[p].at[pl.ds(off, ln)],
                    gsem,
                )

        def drain_gather(p):
            for off, ln in SPLITS:
                pltpu.make_async_copy(
                    table_hbm.at[ixs[p].at[pl.ds(off, ln)]],
                    rws[p].at[pl.ds(off, ln)],
                    gsem,
                ).wait()

        def out_copy(g, p):
            return pltpu.make_async_copy(obs[p], out_hbm.at[row0 + g], osem)

        def compact(p):
            rw = rws[p]
            ob = obs[p]

            def cbody(c2, carry):
                for u in range(2):
                    c = c2 * 2 + u
                    for k in range(D // 16):
                        s = pl.ds(k * 16, 16)
                        ob[c, s] = rw[c, s] * SCALE
                return carry

            lax.fori_loop(0, S // 2, cbody, 0)

        fire(0, 0)
        fire(1, 1)

        def pairstep(h, carry):
            for sub in range(2):
                g = h * 2 + sub

                @pl.when(h > 0)
                def _():
                    out_copy(g - 2, sub).wait()
                drain_gather(sub)
                compact(sub)
                out_copy(g, sub).start()

                @pl.when(g + 2 < rows_per_w)
                def _():
                    fire(g + 2, sub)
            return carry

        lax.fori_loop(0, rows_per_w // 2, pairstep, 0)
        for p in range(2):
            out_copy(rows_per_w - 2 + p, p).wait()

    return emb


def kernel(x, lut):
    R, S = x.shape
    V = lut.shape[0]
    tp = jnp.pad(lut, ((0, 0), (0, DP - D)))
    return _build(R, S, V)(x.astype(jnp.int32), tp)


# speedup vs baseline: 1.7573x; 1.0003x over previous
import functools
import jax
import jax.numpy as jnp
from jax import lax
from jax.experimental import pallas as pl
from jax.experimental.pallas import tpu as pltpu
from jax.experimental.pallas import tpu_sc as plsc

D = 64
DP = 128
SCALE = 8.0
NC, NS = 2, 16
NW = NC * NS
SPLITS = ((0, 104), (104, 96))


@functools.lru_cache(maxsize=None)
def _build(R, S, V):
    rows_per_w = R // NW
    mesh = plsc.VectorSubcoreMesh(core_axis_name="c", subcore_axis_name="s")

    @functools.partial(
        pl.kernel,
        out_type=jax.ShapeDtypeStruct((R, S, D), jnp.float32),
        mesh=mesh,
        compiler_params=pltpu.CompilerParams(
            use_tc_tiling_on_sc=True, needs_layout_passes=False),
        scratch_types=[
            pltpu.VMEM((S,), jnp.int32),
            pltpu.VMEM((S,), jnp.int32),
            pltpu.VMEM((S, DP), jnp.float32),
            pltpu.VMEM((S, DP), jnp.float32),
            pltpu.VMEM((S, D), jnp.float32),
            pltpu.VMEM((S, D), jnp.float32),
            pltpu.SemaphoreType.DMA,
            pltpu.SemaphoreType.DMA,
        ],
    )
    def emb(idx_hbm, table_hbm, out_hbm, ix0, ix1, rw0, rw1, ob0, ob1,
            gsem, osem):
        wid = lax.axis_index("s") * NC + lax.axis_index("c")
        row0 = wid * rows_per_w
        ixs = (ix0, ix1)
        rws = (rw0, rw1)
        obs = (ob0, ob1)

        def fire(g, p):
            pltpu.sync_copy(idx_hbm.at[row0 + g], ixs[p])
            for off, ln in SPLITS:
                pltpu.async_copy(
                    table_hbm.at[ixs[p].at[pl.ds(off, ln)]],
                    rws[p].at[pl.ds(off, ln)],
                    gsem,
                )

        def drain_gather(p):
            for off, ln in SPLITS:
                pltpu.make_async_copy(
                    table_hbm.at[ixs[p].at[pl.ds(off, ln)]],
                    rws[p].at[pl.ds(off, ln)],
                    gsem,
                ).wait()

        def out_copy(g, p):
            return pltpu.make_async_copy(obs[p], out_hbm.at[row0 + g], osem)

        def compact(p):
            rw = rws[p]
            ob = obs[p]

            def cbody(c2, carry):
                for u in range(2):
                    c = c2 * 2 + u
                    for k in range(D // 16):
                        s = pl.ds(k * 16, 16)
                        ob[c, s] = rw[c, s] * SCALE
                return carry

            lax.fori_loop(0, S // 2, cbody, 0)

        fire(0, 0)
        fire(1, 1)

        def pairstep(h, carry):
            for sub in range(2):
                g = h * 2 + sub

                @pl.when(h > 0)
                def _():
                    out_copy(g - 2, sub).wait()
                drain_gather(sub)
                compact(sub)
                out_copy(g, sub).start()

                @pl.when(g + 2 < rows_per_w)
                def _():
                    fire(g + 2, sub)
            return carry

        lax.fori_loop(0, rows_per_w // 2, pairstep, 0)
        for p in range(2):
            out_copy(rows_per_w - 2 + p, p).wait()

    return emb


def kernel(x, lut):
    R, S = x.shape
    V = lut.shape[0]
    tp = jnp.pad(lut.T, ((0, DP - D), (0, 0))).T
    return _build(R, S, V)(x.astype(jnp.int32), tp)
